# Initial kernel scaffold; baseline (speedup 1.0000x reference)
import functools

import jax
import jax.numpy as jnp
from jax import lax
from jax.experimental import pallas as pl
from jax.experimental.pallas import tpu as pltpu
from jax.experimental.pallas import tpu_sc as plsc

_NC = 2
_NS = 16
_NT = _NC * _NS


def _sc_accumulate(src, dst, nf, ef, n_nodes):
    e_total = src.shape[0]
    din = nf.shape[1]
    de = ef.shape[1]
    ept = e_total // _NT
    ch = 80
    nchunk = ept // ch
    rpt = n_nodes // _NS

    mesh = plsc.VectorSubcoreMesh(
        core_axis_name="c", subcore_axis_name="s",
        num_cores=_NC, num_subcores=_NS)

    @functools.partial(
        pl.kernel,
        out_type=[
            jax.ShapeDtypeStruct((_NC, n_nodes, din), jnp.float32),
            jax.ShapeDtypeStruct((_NC, n_nodes, de), jnp.float32),
            jax.ShapeDtypeStruct((_NC, n_nodes, de), jnp.float32),
        ],
        mesh=mesh,
        scratch_types=[
            pltpu.VMEM_SHARED((n_nodes, din), jnp.float32),
            pltpu.VMEM_SHARED((n_nodes, de), jnp.float32),
            pltpu.VMEM_SHARED((n_nodes, de), jnp.float32),
            pltpu.VMEM((80,), jnp.int32),
            pltpu.VMEM((80,), jnp.int32),
            pltpu.VMEM((80, 128), jnp.float32),
            pltpu.VMEM((80, 16), jnp.float32),
            pltpu.VMEM((80, 16), jnp.float32),
            pltpu.VMEM((625, 128), jnp.float32),
            pltpu.VMEM((625, 16), jnp.float32),
            pltpu.SemaphoreType.DMA,
        ],
    )
    def sc_kernel(src_h, dst_h, nf_h, ef_h, za_h, zb_h,
                  a_out, b_out, d_out,
                  a_sh, b_sh, d_sh,
                  src_v, dst_v, rows_v, ef_v, ones_v, sta, stb, sem):
        cid = lax.axis_index("c")
        sid = lax.axis_index("s")

        onehot = jnp.where(lax.iota(jnp.int32, de) == 0,
                           jnp.float32(1.0), jnp.float32(0.0))

        def init_ones(i, carry):
            ones_v[i, :] = onehot
            return carry
        lax.fori_loop(0, ch, init_ones, 0)

        r0 = sid * rpt
        pltpu.sync_copy(za_h, sta)
        pltpu.sync_copy(zb_h, stb)
        pltpu.sync_copy(sta, a_sh.at[pl.ds(r0, rpt)])
        pltpu.sync_copy(stb, b_sh.at[pl.ds(r0, rpt)])
        pltpu.sync_copy(stb, d_sh.at[pl.ds(r0, rpt)])
        plsc.subcore_barrier()

        tid = cid * _NS + sid
        ebase = tid * ept

        def chunk(i, carry):
            base = pl.multiple_of(ebase + i * ch, 8)
            pltpu.sync_copy(src_h.at[pl.ds(base, ch)], src_v)
            pltpu.sync_copy(dst_h.at[pl.ds(base, ch)], dst_v)
            pltpu.async_copy(nf_h.at[src_v], rows_v, sem).wait()
            pltpu.sync_copy(ef_h.at[pl.ds(base, ch)], ef_v)
            pltpu.sync_copy(rows_v, a_sh.at[dst_v], add=True)
            pltpu.sync_copy(ef_v, b_sh.at[dst_v], add=True)
            pltpu.sync_copy(ones_v, d_sh.at[dst_v], add=True)
            return carry
        lax.fori_loop(0, nchunk, chunk, 0)
        plsc.subcore_barrier()

        pltpu.sync_copy(a_sh.at[pl.ds(r0, rpt)], sta)
        pltpu.sync_copy(sta, a_out.at[cid, pl.ds(r0, rpt)])
        pltpu.sync_copy(b_sh.at[pl.ds(r0, rpt)], stb)
        pltpu.sync_copy(stb, b_out.at[cid, pl.ds(r0, rpt)])
        pltpu.sync_copy(d_sh.at[pl.ds(r0, rpt)], stb)
        pltpu.sync_copy(stb, d_out.at[cid, pl.ds(r0, rpt)])

    za = jnp.zeros((rpt, din), jnp.float32)
    zb = jnp.zeros((rpt, de), jnp.float32)
    return sc_kernel(src, dst, nf, ef, za, zb)


def _tc_finish_body(a_ref, b_ref, d_ref, nf_ref, wmh_ref, wme_ref,
                    wa1_ref, wa2_ref, bm_ref, ba_ref, o_ref):
    hi = jax.lax.Precision.HIGHEST
    a = a_ref[0] + a_ref[1]
    b = b_ref[0] + b_ref[1]
    deg = jnp.sum(d_ref[0] + d_ref[1], axis=1, keepdims=True)
    msum = (jnp.dot(a, wmh_ref[...], precision=hi,
                    preferred_element_type=jnp.float32)
            + jnp.dot(b, wme_ref[...], precision=hi,
                      preferred_element_type=jnp.float32)
            + deg * bm_ref[...])
    h_neigh = msum / jnp.maximum(deg, 1.0)
    h = (jnp.dot(nf_ref[...], wa1_ref[...], precision=hi,
                 preferred_element_type=jnp.float32)
         + jnp.dot(h_neigh, wa2_ref[...], precision=hi,
                   preferred_element_type=jnp.float32)
         + ba_ref[...])
    o_ref[...] = jnp.maximum(h, 0.0)


def _tc_finish(a_part, b_part, d_part, nf, wmh_t, wme_t, wa1_t, wa2_t,
               b_msg, b_apply, n_nodes):
    din = nf.shape[1]
    de = b_part.shape[2]
    dout = wmh_t.shape[1]
    rb = 1000
    grid = (n_nodes // rb,)
    return pl.pallas_call(
        _tc_finish_body,
        grid=grid,
        in_specs=[
            pl.BlockSpec((_NC, rb, din), lambda i: (0, i, 0)),
            pl.BlockSpec((_NC, rb, de), lambda i: (0, i, 0)),
            pl.BlockSpec((_NC, rb, de), lambda i: (0, i, 0)),
            pl.BlockSpec((rb, din), lambda i: (i, 0)),
            pl.BlockSpec((din, dout), lambda i: (0, 0)),
            pl.BlockSpec((de, dout), lambda i: (0, 0)),
            pl.BlockSpec((din, dout), lambda i: (0, 0)),
            pl.BlockSpec((dout, dout), lambda i: (0, 0)),
            pl.BlockSpec((1, dout), lambda i: (0, 0)),
            pl.BlockSpec((1, dout), lambda i: (0, 0)),
        ],
        out_specs=pl.BlockSpec((rb, dout), lambda i: (i, 0)),
        out_shape=jax.ShapeDtypeStruct((n_nodes, dout), jnp.float32),
    )(a_part, b_part, d_part, nf, wmh_t, wme_t, wa1_t, wa2_t, b_msg, b_apply)


def kernel(nfeats, efeats, edge_index, W_msg, b_msg, W_apply, b_apply):
    n_nodes = nfeats.shape[0]
    din = nfeats.shape[2]
    de = efeats.shape[2]
    dout = W_msg.shape[0]

    nf = nfeats.reshape(n_nodes, din)
    ef = efeats.reshape(efeats.shape[0], de)
    src = edge_index[0]
    dst = edge_index[1]

    wmh_t = W_msg[:, :din].T
    wme_t = W_msg[:, din:].T
    wa1_t = W_apply[:, :din].T
    wa2_t = W_apply[:, din:].T

    a_part, b_part, d_part = _sc_accumulate(src, dst, nf, ef, n_nodes)
    out = _tc_finish(a_part, b_part, d_part, nf, wmh_t, wme_t, wa1_t, wa2_t,
                     b_msg.reshape(1, dout), b_apply.reshape(1, dout), n_nodes)
    return out.reshape(n_nodes, 1, dout)


# speedup vs baseline: 2.7774x; 2.7774x over previous
import functools

import jax
import jax.numpy as jnp
from jax import lax
from jax.experimental import pallas as pl
from jax.experimental.pallas import tpu as pltpu
from jax.experimental.pallas import tpu_sc as plsc

_NC = 2
_NS = 16


def _sc_accumulate(src, dst, nf0, nf1, ef, n_nodes):
    e_total = src.shape[0]
    dh = nf0.shape[1]
    de = ef.shape[1]
    ept = e_total // _NS
    ch = 80
    nchunk = ept // ch
    rpt = (n_nodes // (8 * _NS)) * 8
    rem = n_nodes - _NS * rpt
    rem0 = _NS * rpt

    mesh = plsc.VectorSubcoreMesh(
        core_axis_name="c", subcore_axis_name="s",
        num_cores=_NC, num_subcores=_NS)

    @functools.partial(
        pl.kernel,
        out_type=[
            jax.ShapeDtypeStruct((_NC, n_nodes, dh), jnp.float32),
            jax.ShapeDtypeStruct((n_nodes, de), jnp.float32),
            jax.ShapeDtypeStruct((n_nodes, de), jnp.float32),
        ],
        mesh=mesh,
        compiler_params=pltpu.CompilerParams(use_tc_tiling_on_sc=False),
        scratch_types=[
            pltpu.VMEM_SHARED((n_nodes, dh), jnp.float32),
            pltpu.VMEM_SHARED((n_nodes, de), jnp.float32),
            pltpu.VMEM((ch,), jnp.int32),
            pltpu.VMEM((ch,), jnp.int32),
            pltpu.VMEM((ch, dh), jnp.float32),
            pltpu.VMEM((ch, de), jnp.float32),
            pltpu.VMEM((rpt, dh), jnp.float32),
            pltpu.VMEM((rpt, de), jnp.float32),
            pltpu.VMEM((rem, dh), jnp.float32),
            pltpu.VMEM((rem, de), jnp.float32),
            pltpu.SemaphoreType.DMA,
        ],
    )
    def sc_kernel(src_h, dst_h, nf0_h, nf1_h, ef_h, za_h, zb_h,
                  a_out, b_out, d_out,
                  a_sh, bd_sh,
                  src_v, dst_v, rows_v, val_v, sta, stb, exa, exb, sem):
        cid = lax.axis_index("c")
        sid = lax.axis_index("s")

        r0 = sid * rpt
        pltpu.sync_copy(za_h, sta)
        pltpu.sync_copy(zb_h, stb)
        pltpu.sync_copy(sta, a_sh.at[pl.ds(r0, rpt)])
        pltpu.sync_copy(stb, bd_sh.at[pl.ds(r0, rpt)])
        if rem:
            @pl.when(sid == 0)
            def _zero_rem():
                pltpu.sync_copy(za_h.at[pl.ds(0, rem)], exa)
                pltpu.sync_copy(zb_h.at[pl.ds(0, rem)], exb)
                pltpu.sync_copy(exa, a_sh.at[pl.ds(rem0, rem)])
                pltpu.sync_copy(exb, bd_sh.at[pl.ds(rem0, rem)])
        plsc.subcore_barrier()

        ebase = sid * ept

        def run_edges(nf_h, load_ef):
            if not load_ef:
                onehot = jnp.where(lax.iota(jnp.int32, de) == 0,
                                   jnp.float32(1.0), jnp.float32(0.0))

                def init_ones(i, carry):
                    val_v[i, :] = onehot
                    return carry
                lax.fori_loop(0, ch, init_ones, 0)

            def chunk(i, carry):
                base = pl.multiple_of(ebase + i * ch, 8)
                pltpu.sync_copy(src_h.at[pl.ds(base, ch)], src_v)
                pltpu.sync_copy(dst_h.at[pl.ds(base, ch)], dst_v)
                pltpu.async_copy(nf_h.at[src_v], rows_v, sem).wait()
                if load_ef:
                    pltpu.sync_copy(ef_h.at[pl.ds(base, ch)], val_v)
                pltpu.sync_copy(rows_v, a_sh.at[dst_v], add=True)
                pltpu.sync_copy(val_v, bd_sh.at[dst_v], add=True)
                return carry
            lax.fori_loop(0, nchunk, chunk, 0)

        @pl.when(cid == 0)
        def _core0():
            run_edges(nf0_h, True)

        @pl.when(cid == 1)
        def _core1():
            run_edges(nf1_h, False)

        plsc.subcore_barrier()

        pltpu.sync_copy(a_sh.at[pl.ds(r0, rpt)], sta)
        pltpu.sync_copy(sta, a_out.at[cid, pl.ds(r0, rpt)])
        pltpu.sync_copy(bd_sh.at[pl.ds(r0, rpt)], stb)

        @pl.when(cid == 0)
        def _wb():
            pltpu.sync_copy(stb, b_out.at[pl.ds(r0, rpt)])

        @pl.when(cid == 1)
        def _wd():
            pltpu.sync_copy(stb, d_out.at[pl.ds(r0, rpt)])

        if rem:
            @pl.when(sid == 0)
            def _write_rem():
                pltpu.sync_copy(a_sh.at[pl.ds(rem0, rem)], exa)
                pltpu.sync_copy(exa, a_out.at[cid, pl.ds(rem0, rem)])
                pltpu.sync_copy(bd_sh.at[pl.ds(rem0, rem)], exb)

                @pl.when(cid == 0)
                def _wbr():
                    pltpu.sync_copy(exb, b_out.at[pl.ds(rem0, rem)])

                @pl.when(cid == 1)
                def _wdr():
                    pltpu.sync_copy(exb, d_out.at[pl.ds(rem0, rem)])

    za = jnp.zeros((rpt, dh), jnp.float32)
    zb = jnp.zeros((rpt, de), jnp.float32)
    return sc_kernel(src, dst, nf0, nf1, ef, za, zb)


def _tc_finish_body(a_ref, b_ref, d_ref, nf_ref, wmh_ref, wme_ref,
                    wa1_ref, wa2_ref, bm_ref, ba_ref, o_ref):
    hi = jax.lax.Precision.HIGHEST
    dh = a_ref.shape[2]
    deg = jnp.sum(d_ref[...], axis=1, keepdims=True)
    msum = (jnp.dot(a_ref[0], wmh_ref[:dh], precision=hi,
                    preferred_element_type=jnp.float32)
            + jnp.dot(a_ref[1], wmh_ref[dh:], precision=hi,
                      preferred_element_type=jnp.float32)
            + jnp.dot(b_ref[...], wme_ref[...], precision=hi,
                      preferred_element_type=jnp.float32)
            + deg * bm_ref[...])
    h_neigh = msum / jnp.maximum(deg, 1.0)
    h = (jnp.dot(nf_ref[...], wa1_ref[...], precision=hi,
                 preferred_element_type=jnp.float32)
         + jnp.dot(h_neigh, wa2_ref[...], precision=hi,
                   preferred_element_type=jnp.float32)
         + ba_ref[...])
    o_ref[...] = jnp.maximum(h, 0.0)


def _tc_finish(a_part, b_part, d_part, nf, wmh_t, wme_t, wa1_t, wa2_t,
               b_msg, b_apply, n_nodes):
    din = nf.shape[1]
    dh = a_part.shape[2]
    de = b_part.shape[1]
    dout = wmh_t.shape[1]
    rb = 1000
    grid = (n_nodes // rb,)
    return pl.pallas_call(
        _tc_finish_body,
        grid=grid,
        in_specs=[
            pl.BlockSpec((_NC, rb, dh), lambda i: (0, i, 0)),
            pl.BlockSpec((rb, de), lambda i: (i, 0)),
            pl.BlockSpec((rb, de), lambda i: (i, 0)),
            pl.BlockSpec((rb, din), lambda i: (i, 0)),
            pl.BlockSpec((din, dout), lambda i: (0, 0)),
            pl.BlockSpec((de, dout), lambda i: (0, 0)),
            pl.BlockSpec((din, dout), lambda i: (0, 0)),
            pl.BlockSpec((dout, dout), lambda i: (0, 0)),
            pl.BlockSpec((1, dout), lambda i: (0, 0)),
            pl.BlockSpec((1, dout), lambda i: (0, 0)),
        ],
        out_specs=pl.BlockSpec((rb, dout), lambda i: (i, 0)),
        out_shape=jax.ShapeDtypeStruct((n_nodes, dout), jnp.float32),
    )(a_part, b_part, d_part, nf, wmh_t, wme_t, wa1_t, wa2_t, b_msg, b_apply)


def kernel(nfeats, efeats, edge_index, W_msg, b_msg, W_apply, b_apply):
    n_nodes = nfeats.shape[0]
    din = nfeats.shape[2]
    de = efeats.shape[2]
    dout = W_msg.shape[0]
    dh = din // 2

    nf = nfeats.reshape(n_nodes, din)
    ef = efeats.reshape(efeats.shape[0], de)
    src = edge_index[0]
    dst = edge_index[1]

    wmh_t = W_msg[:, :din].T
    wme_t = W_msg[:, din:].T
    wa1_t = W_apply[:, :din].T
    wa2_t = W_apply[:, din:].T

    a_part, b_part, d_part = _sc_accumulate(
        src, dst, nf[:, :dh], nf[:, dh:], ef, n_nodes)
    out = _tc_finish(a_part, b_part, d_part, nf, wmh_t, wme_t, wa1_t, wa2_t,
                     b_msg.reshape(1, dout), b_apply.reshape(1, dout), n_nodes)
    return out.reshape(n_nodes, 1, dout)


# speedup vs baseline: 6.2562x; 2.2525x over previous
import functools

import jax
import jax.numpy as jnp
from jax import lax
from jax.experimental import pallas as pl
from jax.experimental.pallas import tpu as pltpu
from jax.experimental.pallas import tpu_sc as plsc

_NC = 2
_NS = 16


def _sc_accumulate(src, dst, nf0, nf1, ef, n_nodes):
    e_total = src.shape[0]
    dh = nf0.shape[1]
    de = ef.shape[1]
    ept = e_total // _NS
    ch = 80
    sb = 5
    rows_pt = ept // ch
    nsc = rows_pt // sb
    rpt = (n_nodes // (8 * _NS)) * 8
    rem = n_nodes - _NS * rpt
    rem0 = _NS * rpt
    nst = 3
    spt = rpt // nst

    mesh = plsc.VectorSubcoreMesh(
        core_axis_name="c", subcore_axis_name="s",
        num_cores=_NC, num_subcores=_NS)

    @functools.partial(
        pl.kernel,
        out_type=[
            jax.ShapeDtypeStruct((_NC, n_nodes, dh), jnp.float32),
            jax.ShapeDtypeStruct((n_nodes, de), jnp.float32),
            jax.ShapeDtypeStruct((n_nodes, de), jnp.float32),
        ],
        mesh=mesh,
        compiler_params=pltpu.CompilerParams(use_tc_tiling_on_sc=False),
        scratch_types=[
            pltpu.VMEM_SHARED((n_nodes, dh), jnp.float32),
            pltpu.VMEM_SHARED((n_nodes, de), jnp.float32),
            pltpu.VMEM((2, sb, ch), jnp.int32),
            pltpu.VMEM((2, sb, ch), jnp.int32),
            pltpu.VMEM((sb, ch, dh), jnp.float32),
            pltpu.VMEM((sb * ch, de), jnp.float32),
            pltpu.VMEM((ch, de), jnp.float32),
            pltpu.VMEM((spt, dh), jnp.float32),
            pltpu.VMEM((spt, de), jnp.float32),
            pltpu.VMEM((rem, dh), jnp.float32),
            pltpu.VMEM((rem, de), jnp.float32),
            pltpu.SemaphoreType.DMA,
            pltpu.SemaphoreType.DMA,
            pltpu.SemaphoreType.DMA,
        ],
    )
    def sc_kernel(src_h, dst_h, nf0_h, nf1_h, ef_h, za_h, zb_h,
                  a_out, b_out, d_out,
                  a_sh, bd_sh,
                  src_i, dst_i, rows_v, val_v, ones_v, sta, stb, exa, exb,
                  sem_i, sem_g, sem_s):
        cid = lax.axis_index("c")
        sid = lax.axis_index("s")

        r0 = sid * rpt
        pltpu.sync_copy(za_h, sta)
        pltpu.sync_copy(zb_h, stb)
        for k in range(nst):
            pltpu.sync_copy(sta, a_sh.at[pl.ds(r0 + k * spt, spt)])
            pltpu.sync_copy(stb, bd_sh.at[pl.ds(r0 + k * spt, spt)])
        if rem:
            @pl.when(sid == 0)
            def _zero_rem():
                pltpu.sync_copy(za_h.at[pl.ds(0, rem)], exa)
                pltpu.sync_copy(zb_h.at[pl.ds(0, rem)], exb)
                pltpu.sync_copy(exa, a_sh.at[pl.ds(rem0, rem)])
                pltpu.sync_copy(exb, bd_sh.at[pl.ds(rem0, rem)])
        plsc.subcore_barrier()

        rowbase = sid * rows_pt

        def run_edges(nf_h, load_ef):
            if not load_ef:
                onehot = jnp.where(lax.iota(jnp.int32, de) == 0,
                                   jnp.float32(1.0), jnp.float32(0.0))

                def init_ones(i, carry):
                    ones_v[i, :] = onehot
                    return carry
                lax.fori_loop(0, ch, init_ones, 0)

            def idx_copies(row0, ring, make_only):
                mk = pltpu.make_async_copy
                ds = []
                for b in range(sb):
                    e0 = pl.multiple_of((row0 + b) * ch, 8)
                    ds.append(mk(src_h.at[pl.ds(e0, ch)],
                                 src_i.at[ring, b], sem_i))
                    ds.append(mk(dst_h.at[pl.ds(e0, ch)],
                                 dst_i.at[ring, b], sem_i))
                if not make_only:
                    for d in ds:
                        d.start()
                return ds

            def fire_idx(row0, ring):
                idx_copies(row0, ring, False)

            fire_idx(rowbase, 0)

            def superchunk(g, carry):
                r = lax.rem(g, 2)
                row0 = rowbase + g * sb
                for d in idx_copies(row0, r, True):
                    d.wait()
                gd = [pltpu.async_copy(nf_h.at[src_i.at[r, b]],
                                       rows_v.at[b], sem_g)
                      for b in range(sb)]
                if load_ef:
                    gd.append(pltpu.async_copy(
                        ef_h.at[pl.ds(row0 * ch, sb * ch)], val_v, sem_g))

                @pl.when(g < nsc - 1)
                def _prefetch():
                    fire_idx(row0 + sb, 1 - r)

                for d in gd:
                    d.wait()
                sd = []
                for b in range(sb):
                    sd.append(pltpu.async_copy(
                        rows_v.at[b], a_sh.at[dst_i.at[r, b]], sem_s,
                        add=True))
                    vsrc = (val_v.at[pl.ds(b * ch, ch)] if load_ef
                            else ones_v)
                    sd.append(pltpu.async_copy(
                        vsrc, bd_sh.at[dst_i.at[r, b]], sem_s, add=True))
                for d in sd:
                    d.wait()
                return carry
            lax.fori_loop(0, nsc, superchunk, 0)

        @pl.when(cid == 0)
        def _core0():
            run_edges(nf0_h, True)

        @pl.when(cid == 1)
        def _core1():
            run_edges(nf1_h, False)

        plsc.subcore_barrier()

        for k in range(nst):
            rk = r0 + k * spt
            pltpu.sync_copy(a_sh.at[pl.ds(rk, spt)], sta)
            pltpu.sync_copy(sta, a_out.at[cid, pl.ds(rk, spt)])
            pltpu.sync_copy(bd_sh.at[pl.ds(rk, spt)], stb)

            @pl.when(cid == 0)
            def _wb():
                pltpu.sync_copy(stb, b_out.at[pl.ds(rk, spt)])

            @pl.when(cid == 1)
            def _wd():
                pltpu.sync_copy(stb, d_out.at[pl.ds(rk, spt)])

        if rem:
            @pl.when(sid == 0)
            def _write_rem():
                pltpu.sync_copy(a_sh.at[pl.ds(rem0, rem)], exa)
                pltpu.sync_copy(exa, a_out.at[cid, pl.ds(rem0, rem)])
                pltpu.sync_copy(bd_sh.at[pl.ds(rem0, rem)], exb)

                @pl.when(cid == 0)
                def _wbr():
                    pltpu.sync_copy(exb, b_out.at[pl.ds(rem0, rem)])

                @pl.when(cid == 1)
                def _wdr():
                    pltpu.sync_copy(exb, d_out.at[pl.ds(rem0, rem)])

    za = jnp.zeros((spt, dh), jnp.float32)
    zb = jnp.zeros((spt, de), jnp.float32)
    return sc_kernel(src, dst, nf0, nf1, ef, za, zb)


def _tc_finish_body(a_ref, b_ref, d_ref, nf_ref, wmh_ref, wme_ref,
                    wa1_ref, wa2_ref, bm_ref, ba_ref, o_ref):
    hi = jax.lax.Precision.HIGHEST
    dh = a_ref.shape[2]
    deg = jnp.sum(d_ref[...], axis=1, keepdims=True)
    msum = (jnp.dot(a_ref[0], wmh_ref[:dh], precision=hi,
                    preferred_element_type=jnp.float32)
            + jnp.dot(a_ref[1], wmh_ref[dh:], precision=hi,
                      preferred_element_type=jnp.float32)
            + jnp.dot(b_ref[...], wme_ref[...], precision=hi,
                      preferred_element_type=jnp.float32)
            + deg * bm_ref[...])
    h_neigh = msum / jnp.maximum(deg, 1.0)
    h = (jnp.dot(nf_ref[...], wa1_ref[...], precision=hi,
                 preferred_element_type=jnp.float32)
         + jnp.dot(h_neigh, wa2_ref[...], precision=hi,
                   preferred_element_type=jnp.float32)
         + ba_ref[...])
    o_ref[...] = jnp.maximum(h, 0.0)


def _tc_finish(a_part, b_part, d_part, nf, wmh_t, wme_t, wa1_t, wa2_t,
               b_msg, b_apply, n_nodes):
    din = nf.shape[1]
    dh = a_part.shape[2]
    de = b_part.shape[1]
    dout = wmh_t.shape[1]
    rb = 1000
    grid = (n_nodes // rb,)
    return pl.pallas_call(
        _tc_finish_body,
        grid=grid,
        in_specs=[
            pl.BlockSpec((_NC, rb, dh), lambda i: (0, i, 0)),
            pl.BlockSpec((rb, de), lambda i: (i, 0)),
            pl.BlockSpec((rb, de), lambda i: (i, 0)),
            pl.BlockSpec((rb, din), lambda i: (i, 0)),
            pl.BlockSpec((din, dout), lambda i: (0, 0)),
            pl.BlockSpec((de, dout), lambda i: (0, 0)),
            pl.BlockSpec((din, dout), lambda i: (0, 0)),
            pl.BlockSpec((dout, dout), lambda i: (0, 0)),
            pl.BlockSpec((1, dout), lambda i: (0, 0)),
            pl.BlockSpec((1, dout), lambda i: (0, 0)),
        ],
        out_specs=pl.BlockSpec((rb, dout), lambda i: (i, 0)),
        out_shape=jax.ShapeDtypeStruct((n_nodes, dout), jnp.float32),
    )(a_part, b_part, d_part, nf, wmh_t, wme_t, wa1_t, wa2_t, b_msg, b_apply)


def kernel(nfeats, efeats, edge_index, W_msg, b_msg, W_apply, b_apply):
    n_nodes = nfeats.shape[0]
    din = nfeats.shape[2]
    de = efeats.shape[2]
    dout = W_msg.shape[0]
    dh = din // 2

    nf = nfeats.reshape(n_nodes, din)
    ef = efeats.reshape(efeats.shape[0], de)
    src = edge_index[0]
    dst = edge_index[1]

    wmh_t = W_msg[:, :din].T
    wme_t = W_msg[:, din:].T
    wa1_t = W_apply[:, :din].T
    wa2_t = W_apply[:, din:].T

    a_part, b_part, d_part = _sc_accumulate(
        src, dst, nf[:, :dh], nf[:, dh:], ef, n_nodes)
    out = _tc_finish(a_part, b_part, d_part, nf, wmh_t, wme_t, wa1_t, wa2_t,
                     b_msg.reshape(1, dout), b_apply.reshape(1, dout), n_nodes)
    return out.reshape(n_nodes, 1, dout)


# speedup vs baseline: 6.9360x; 1.1087x over previous
import functools

import jax
import jax.numpy as jnp
from jax import lax
from jax.experimental import pallas as pl
from jax.experimental.pallas import tpu as pltpu
from jax.experimental.pallas import tpu_sc as plsc

_NC = 2
_NS = 16


def _sc_accumulate(src, dst, nf0, nf1, ef, n_nodes):
    e_total = src.shape[0]
    dh = nf0.shape[1]
    de = ef.shape[1]
    ept = e_total // _NS
    ch = 80
    sb = 5
    rows_pt = ept // ch
    nsc = rows_pt // sb
    rpt = (n_nodes // (8 * _NS)) * 8
    rem = n_nodes - _NS * rpt
    rem0 = _NS * rpt
    nst = 6
    spt = rpt // nst

    mesh = plsc.VectorSubcoreMesh(
        core_axis_name="c", subcore_axis_name="s",
        num_cores=_NC, num_subcores=_NS)

    @functools.partial(
        pl.kernel,
        out_type=[
            jax.ShapeDtypeStruct((_NC, n_nodes, dh), jnp.float32),
            jax.ShapeDtypeStruct((n_nodes, de), jnp.float32),
            jax.ShapeDtypeStruct((n_nodes, de), jnp.float32),
        ],
        mesh=mesh,
        compiler_params=pltpu.CompilerParams(use_tc_tiling_on_sc=False),
        scratch_types=[
            pltpu.VMEM_SHARED((n_nodes, dh), jnp.float32),
            pltpu.VMEM_SHARED((n_nodes, de), jnp.float32),
            pltpu.VMEM((2, sb, ch), jnp.int32),
            pltpu.VMEM((2, sb, ch), jnp.int32),
            pltpu.VMEM((2, sb, ch), jnp.int32),
            pltpu.VMEM((2, sb, ch, dh), jnp.float32),
            pltpu.VMEM((2, sb * ch, de), jnp.float32),
            pltpu.VMEM((ch, de), jnp.float32),
            pltpu.VMEM((spt, dh), jnp.float32),
            pltpu.VMEM((spt, de), jnp.float32),
            pltpu.VMEM((rem, dh), jnp.float32),
            pltpu.VMEM((rem, de), jnp.float32),
            pltpu.SemaphoreType.DMA,
            pltpu.SemaphoreType.DMA,
            pltpu.SemaphoreType.DMA,
            pltpu.SemaphoreType.DMA,
            pltpu.SemaphoreType.DMA,
        ],
    )
    def sc_kernel(src_h, dst_h, nf0_h, nf1_h, ef_h, za_h, zb_h,
                  a_out, b_out, d_out,
                  a_sh, bd_sh,
                  src_i, dst_i, dst_s, rows_v, val_v, ones_v, sta, stb,
                  exa, exb,
                  sem_i0, sem_i1, sem_g, sem_s0, sem_s1):
        cid = lax.axis_index("c")
        sid = lax.axis_index("s")

        r0 = sid * rpt
        pltpu.sync_copy(za_h, sta)
        pltpu.sync_copy(zb_h, stb)
        for k in range(nst):
            pltpu.sync_copy(sta, a_sh.at[pl.ds(r0 + k * spt, spt)])
            pltpu.sync_copy(stb, bd_sh.at[pl.ds(r0 + k * spt, spt)])
        if rem:
            @pl.when(sid == 0)
            def _zero_rem():
                pltpu.sync_copy(za_h.at[pl.ds(0, rem)], exa)
                pltpu.sync_copy(zb_h.at[pl.ds(0, rem)], exb)
                pltpu.sync_copy(exa, a_sh.at[pl.ds(rem0, rem)])
                pltpu.sync_copy(exb, bd_sh.at[pl.ds(rem0, rem)])
        plsc.subcore_barrier()

        rowbase = sid * rows_pt

        def run_edges(nf_h, load_ef):
            if not load_ef:
                onehot = jnp.where(lax.iota(jnp.int32, de) == 0,
                                   jnp.float32(1.0), jnp.float32(0.0))

                def init_ones(i, carry):
                    ones_v[i, :] = onehot
                    return carry
                lax.fori_loop(0, ch, init_ones, 0)

            sem_i = (sem_i0, sem_i1)
            sem_s = (sem_s0, sem_s1)

            def idx_copies(row0, p, make_only):
                mk = pltpu.make_async_copy
                ds = []
                for b in range(sb):
                    e0 = pl.multiple_of((row0 + b) * ch, 8)
                    ds.append(mk(src_h.at[pl.ds(e0, ch)],
                                 src_i.at[p, b], sem_i[p]))
                    ds.append(mk(dst_h.at[pl.ds(e0, ch)],
                                 dst_i.at[p, b], sem_i[p]))
                if not make_only:
                    for d in ds:
                        d.start()
                return ds

            def scatter_copies(p, make_only):
                mk = pltpu.make_async_copy
                ds = []
                for b in range(sb):
                    ds.append(mk(rows_v.at[p, b], a_sh.at[dst_s.at[p, b]],
                                 sem_s[p]))
                    vsrc = (val_v.at[p, pl.ds(b * ch, ch)] if load_ef
                            else ones_v)
                    ds.append(mk(vsrc, bd_sh.at[dst_s.at[p, b]], sem_s[p]))
                if not make_only:
                    for d in ds:
                        d.start(add=True)
                return ds

            idx_copies(rowbase, 0, False)
            idx_copies(rowbase + sb, 1, False)

            def section(s, p):
                row0 = rowbase + s * sb
                @pl.when(s >= 2)
                def _drain_scatters():
                    for d in scatter_copies(p, True):
                        d.wait()
                for d in idx_copies(row0, p, True):
                    d.wait()
                for b in range(sb):
                    for j in range(ch // 16):
                        dst_s[p, b, pl.ds(j * 16, 16)] = (
                            dst_i[p, b, pl.ds(j * 16, 16)])
                gd = [pltpu.async_copy(nf_h.at[src_i.at[p, b]],
                                       rows_v.at[p, b], sem_g)
                      for b in range(sb)]
                if load_ef:
                    gd.append(pltpu.async_copy(
                        ef_h.at[pl.ds(row0 * ch, sb * ch)], val_v.at[p],
                        sem_g))
                @pl.when(s < nsc - 1)
                def _prefetch():
                    idx_copies(row0 + sb, 1 - p, False)

                for d in gd:
                    d.wait()
                scatter_copies(p, False)

            def pairbody(k, carry):
                section(2 * k, 0)
                section(2 * k + 1, 1)
                return carry
            lax.fori_loop(0, nsc // 2, pairbody, 0)
            for p in range(2):
                for d in scatter_copies(p, True):
                    d.wait()

        @pl.when(cid == 0)
        def _core0():
            run_edges(nf0_h, True)

        @pl.when(cid == 1)
        def _core1():
            run_edges(nf1_h, False)

        plsc.subcore_barrier()

        for k in range(nst):
            rk = r0 + k * spt
            pltpu.sync_copy(a_sh.at[pl.ds(rk, spt)], sta)
            pltpu.sync_copy(sta, a_out.at[cid, pl.ds(rk, spt)])
            pltpu.sync_copy(bd_sh.at[pl.ds(rk, spt)], stb)

            @pl.when(cid == 0)
            def _wb():
                pltpu.sync_copy(stb, b_out.at[pl.ds(rk, spt)])

            @pl.when(cid == 1)
            def _wd():
                pltpu.sync_copy(stb, d_out.at[pl.ds(rk, spt)])

        if rem:
            @pl.when(sid == 0)
            def _write_rem():
                pltpu.sync_copy(a_sh.at[pl.ds(rem0, rem)], exa)
                pltpu.sync_copy(exa, a_out.at[cid, pl.ds(rem0, rem)])
                pltpu.sync_copy(bd_sh.at[pl.ds(rem0, rem)], exb)

                @pl.when(cid == 0)
                def _wbr():
                    pltpu.sync_copy(exb, b_out.at[pl.ds(rem0, rem)])

                @pl.when(cid == 1)
                def _wdr():
                    pltpu.sync_copy(exb, d_out.at[pl.ds(rem0, rem)])

    za = jnp.zeros((spt, dh), jnp.float32)
    zb = jnp.zeros((spt, de), jnp.float32)
    return sc_kernel(src, dst, nf0, nf1, ef, za, zb)


def _tc_finish_body(a_ref, b_ref, d_ref, nf_ref, wmh_ref, wme_ref,
                    wa1_ref, wa2_ref, bm_ref, ba_ref, o_ref):
    hi = jax.lax.Precision.HIGHEST
    dh = a_ref.shape[2]
    deg = jnp.sum(d_ref[...], axis=1, keepdims=True)
    msum = (jnp.dot(a_ref[0], wmh_ref[:dh], precision=hi,
                    preferred_element_type=jnp.float32)
            + jnp.dot(a_ref[1], wmh_ref[dh:], precision=hi,
                      preferred_element_type=jnp.float32)
            + jnp.dot(b_ref[...], wme_ref[...], precision=hi,
                      preferred_element_type=jnp.float32)
            + deg * bm_ref[...])
    h_neigh = msum / jnp.maximum(deg, 1.0)
    h = (jnp.dot(nf_ref[...], wa1_ref[...], precision=hi,
                 preferred_element_type=jnp.float32)
         + jnp.dot(h_neigh, wa2_ref[...], precision=hi,
                   preferred_element_type=jnp.float32)
         + ba_ref[...])
    o_ref[...] = jnp.maximum(h, 0.0)


def _tc_finish(a_part, b_part, d_part, nf, wmh_t, wme_t, wa1_t, wa2_t,
               b_msg, b_apply, n_nodes):
    din = nf.shape[1]
    dh = a_part.shape[2]
    de = b_part.shape[1]
    dout = wmh_t.shape[1]
    rb = 1000
    grid = (n_nodes // rb,)
    return pl.pallas_call(
        _tc_finish_body,
        grid=grid,
        in_specs=[
            pl.BlockSpec((_NC, rb, dh), lambda i: (0, i, 0)),
            pl.BlockSpec((rb, de), lambda i: (i, 0)),
            pl.BlockSpec((rb, de), lambda i: (i, 0)),
            pl.BlockSpec((rb, din), lambda i: (i, 0)),
            pl.BlockSpec((din, dout), lambda i: (0, 0)),
            pl.BlockSpec((de, dout), lambda i: (0, 0)),
            pl.BlockSpec((din, dout), lambda i: (0, 0)),
            pl.BlockSpec((dout, dout), lambda i: (0, 0)),
            pl.BlockSpec((1, dout), lambda i: (0, 0)),
            pl.BlockSpec((1, dout), lambda i: (0, 0)),
        ],
        out_specs=pl.BlockSpec((rb, dout), lambda i: (i, 0)),
        out_shape=jax.ShapeDtypeStruct((n_nodes, dout), jnp.float32),
    )(a_part, b_part, d_part, nf, wmh_t, wme_t, wa1_t, wa2_t, b_msg, b_apply)


def kernel(nfeats, efeats, edge_index, W_msg, b_msg, W_apply, b_apply):
    n_nodes = nfeats.shape[0]
    din = nfeats.shape[2]
    de = efeats.shape[2]
    dout = W_msg.shape[0]
    dh = din // 2

    nf = nfeats.reshape(n_nodes, din)
    ef = efeats.reshape(efeats.shape[0], de)
    src = edge_index[0]
    dst = edge_index[1]

    wmh_t = W_msg[:, :din].T
    wme_t = W_msg[:, din:].T
    wa1_t = W_apply[:, :din].T
    wa2_t = W_apply[:, din:].T

    a_part, b_part, d_part = _sc_accumulate(
        src, dst, nf[:, :dh], nf[:, dh:], ef, n_nodes)
    out = _tc_finish(a_part, b_part, d_part, nf, wmh_t, wme_t, wa1_t, wa2_t,
                     b_msg.reshape(1, dout), b_apply.reshape(1, dout), n_nodes)
    return out.reshape(n_nodes, 1, dout)


# speedup vs baseline: 6.9665x; 1.0044x over previous
import functools

import jax
import jax.numpy as jnp
from jax import lax
from jax.experimental import pallas as pl
from jax.experimental.pallas import tpu as pltpu
from jax.experimental.pallas import tpu_sc as plsc

_NC = 2
_NS = 16


def _sc_accumulate(src, dst, nf0, nf1, ef, n_nodes):
    e_total = src.shape[0]
    dh = nf0.shape[1]
    de = ef.shape[1]
    ept = e_total // _NS
    ch = 80
    sb = 5
    rows_pt = ept // ch
    nsc = rows_pt // sb
    rpt = (n_nodes // (8 * _NS)) * 8
    rem = n_nodes - _NS * rpt
    rem0 = _NS * rpt
    nst = 6
    spt = rpt // nst

    mesh = plsc.VectorSubcoreMesh(
        core_axis_name="c", subcore_axis_name="s",
        num_cores=_NC, num_subcores=_NS)

    @functools.partial(
        pl.kernel,
        out_type=[
            jax.ShapeDtypeStruct((n_nodes, dh), jnp.float32),
            jax.ShapeDtypeStruct((n_nodes, dh), jnp.float32),
            jax.ShapeDtypeStruct((n_nodes, de), jnp.float32),
            jax.ShapeDtypeStruct((n_nodes, de), jnp.float32),
        ],
        mesh=mesh,
        compiler_params=pltpu.CompilerParams(use_tc_tiling_on_sc=False),
        scratch_types=[
            pltpu.VMEM_SHARED((n_nodes, dh), jnp.float32),
            pltpu.VMEM_SHARED((n_nodes, de), jnp.float32),
            pltpu.VMEM((2, sb, ch), jnp.int32),
            pltpu.VMEM((2, sb, ch), jnp.int32),
            pltpu.VMEM((2, sb, ch), jnp.int32),
            pltpu.VMEM((2, sb, ch, dh), jnp.float32),
            pltpu.VMEM((2, sb * ch, de), jnp.float32),
            pltpu.VMEM((ch, de), jnp.float32),
            pltpu.VMEM((spt, dh), jnp.float32),
            pltpu.VMEM((spt, de), jnp.float32),
            pltpu.VMEM((rem, dh), jnp.float32),
            pltpu.VMEM((rem, de), jnp.float32),
            pltpu.SemaphoreType.DMA,
            pltpu.SemaphoreType.DMA,
            pltpu.SemaphoreType.DMA,
            pltpu.SemaphoreType.DMA,
            pltpu.SemaphoreType.DMA,
        ],
    )
    def sc_kernel(src_h, dst_h, nf0_h, nf1_h, ef_h, za_h, zb_h,
                  a0_out, a1_out, b_out, d_out,
                  a_sh, bd_sh,
                  src_i, dst_i, dst_s, rows_v, val_v, ones_v, sta, stb,
                  exa, exb,
                  sem_i0, sem_i1, sem_g, sem_s0, sem_s1):
        cid = lax.axis_index("c")
        sid = lax.axis_index("s")

        r0 = sid * rpt
        pltpu.sync_copy(za_h, sta)
        pltpu.sync_copy(zb_h, stb)
        for k in range(nst):
            pltpu.sync_copy(sta, a_sh.at[pl.ds(r0 + k * spt, spt)])
            pltpu.sync_copy(stb, bd_sh.at[pl.ds(r0 + k * spt, spt)])
        if rem:
            @pl.when(sid == 0)
            def _zero_rem():
                pltpu.sync_copy(za_h.at[pl.ds(0, rem)], exa)
                pltpu.sync_copy(zb_h.at[pl.ds(0, rem)], exb)
                pltpu.sync_copy(exa, a_sh.at[pl.ds(rem0, rem)])
                pltpu.sync_copy(exb, bd_sh.at[pl.ds(rem0, rem)])
        plsc.subcore_barrier()

        rowbase = sid * rows_pt

        def run_edges(nf_h, load_ef):
            if not load_ef:
                onehot = jnp.where(lax.iota(jnp.int32, de) == 0,
                                   jnp.float32(1.0), jnp.float32(0.0))

                def init_ones(i, carry):
                    ones_v[i, :] = onehot
                    return carry
                lax.fori_loop(0, ch, init_ones, 0)

            sem_i = (sem_i0, sem_i1)
            sem_s = (sem_s0, sem_s1)

            def idx_copies(row0, p, make_only):
                mk = pltpu.make_async_copy
                ds = []
                for b in range(sb):
                    e0 = pl.multiple_of((row0 + b) * ch, 8)
                    ds.append(mk(src_h.at[pl.ds(e0, ch)],
                                 src_i.at[p, b], sem_i[p]))
                    ds.append(mk(dst_h.at[pl.ds(e0, ch)],
                                 dst_i.at[p, b], sem_i[p]))
                if not make_only:
                    for d in ds:
                        d.start()
                return ds

            def scatter_copies(p, make_only):
                mk = pltpu.make_async_copy
                ds = []
                for b in range(sb):
                    ds.append(mk(rows_v.at[p, b], a_sh.at[dst_s.at[p, b]],
                                 sem_s[p]))
                    vsrc = (val_v.at[p, pl.ds(b * ch, ch)] if load_ef
                            else ones_v)
                    ds.append(mk(vsrc, bd_sh.at[dst_s.at[p, b]], sem_s[p]))
                if not make_only:
                    for d in ds:
                        d.start(add=True)
                return ds

            idx_copies(rowbase, 0, False)
            idx_copies(rowbase + sb, 1, False)

            def section(s, p):
                row0 = rowbase + s * sb
                @pl.when(s >= 2)
                def _drain_scatters():
                    for d in scatter_copies(p, True):
                        d.wait()
                for d in idx_copies(row0, p, True):
                    d.wait()
                for b in range(sb):
                    for j in range(ch // 16):
                        dst_s[p, b, pl.ds(j * 16, 16)] = (
                            dst_i[p, b, pl.ds(j * 16, 16)])
                gd = [pltpu.async_copy(nf_h.at[src_i.at[p, b]],
                                       rows_v.at[p, b], sem_g)
                      for b in range(sb)]
                if load_ef:
                    gd.append(pltpu.async_copy(
                        ef_h.at[pl.ds(row0 * ch, sb * ch)], val_v.at[p],
                        sem_g))
                @pl.when(s < nsc - 1)
                def _prefetch():
                    idx_copies(row0 + sb, 1 - p, False)

                for d in gd:
                    d.wait()
                scatter_copies(p, False)

            def pairbody(k, carry):
                section(2 * k, 0)
                section(2 * k + 1, 1)
                return carry
            lax.fori_loop(0, nsc // 2, pairbody, 0)
            for p in range(2):
                for d in scatter_copies(p, True):
                    d.wait()

        @pl.when(cid == 0)
        def _core0():
            run_edges(nf0_h, True)

        @pl.when(cid == 1)
        def _core1():
            run_edges(nf1_h, False)

        plsc.subcore_barrier()

        for k in range(nst):
            rk = r0 + k * spt
            pltpu.sync_copy(a_sh.at[pl.ds(rk, spt)], sta)
            pltpu.sync_copy(bd_sh.at[pl.ds(rk, spt)], stb)

            @pl.when(cid == 0)
            def _w0():
                pltpu.sync_copy(sta, a0_out.at[pl.ds(rk, spt)])
                pltpu.sync_copy(stb, b_out.at[pl.ds(rk, spt)])

            @pl.when(cid == 1)
            def _w1():
                pltpu.sync_copy(sta, a1_out.at[pl.ds(rk, spt)])
                pltpu.sync_copy(stb, d_out.at[pl.ds(rk, spt)])

        if rem:
            @pl.when(sid == 0)
            def _write_rem():
                pltpu.sync_copy(a_sh.at[pl.ds(rem0, rem)], exa)
                pltpu.sync_copy(bd_sh.at[pl.ds(rem0, rem)], exb)

                @pl.when(cid == 0)
                def _wbr():
                    pltpu.sync_copy(exa, a0_out.at[pl.ds(rem0, rem)])
                    pltpu.sync_copy(exb, b_out.at[pl.ds(rem0, rem)])

                @pl.when(cid == 1)
                def _wdr():
                    pltpu.sync_copy(exa, a1_out.at[pl.ds(rem0, rem)])
                    pltpu.sync_copy(exb, d_out.at[pl.ds(rem0, rem)])

    za = jnp.zeros((spt, dh), jnp.float32)
    zb = jnp.zeros((spt, de), jnp.float32)
    return sc_kernel(src, dst, nf0, nf1, ef, za, zb)


def _tc_finish_body(a0_ref, a1_ref, b_ref, d_ref, nf_ref, wmh_ref, wme_ref,
                    wa1_ref, wa2_ref, bm_ref, ba_ref, o_ref):
    hi = jax.lax.Precision.HIGHEST
    dh = a0_ref.shape[1]
    deg = jnp.sum(d_ref[...], axis=1, keepdims=True)
    msum = (jnp.dot(a0_ref[...], wmh_ref[:dh], precision=hi,
                    preferred_element_type=jnp.float32)
            + jnp.dot(a1_ref[...], wmh_ref[dh:], precision=hi,
                      preferred_element_type=jnp.float32)
            + jnp.dot(b_ref[...], wme_ref[...], precision=hi,
                      preferred_element_type=jnp.float32)
            + deg * bm_ref[...])
    h_neigh = msum / jnp.maximum(deg, 1.0)
    h = (jnp.dot(nf_ref[...], wa1_ref[...], precision=hi,
                 preferred_element_type=jnp.float32)
         + jnp.dot(h_neigh, wa2_ref[...], precision=hi,
                   preferred_element_type=jnp.float32)
         + ba_ref[...])
    o_ref[...] = jnp.maximum(h, 0.0)


def _tc_finish(a0, a1, b_part, d_part, nf, wmh_t, wme_t, wa1_t, wa2_t,
               b_msg, b_apply, n_nodes):
    din = nf.shape[1]
    dh = a0.shape[1]
    de = b_part.shape[1]
    dout = wmh_t.shape[1]
    rb = 1000
    grid = (n_nodes // rb,)
    return pl.pallas_call(
        _tc_finish_body,
        grid=grid,
        in_specs=[
            pl.BlockSpec((rb, dh), lambda i: (i, 0)),
            pl.BlockSpec((rb, dh), lambda i: (i, 0)),
            pl.BlockSpec((rb, de), lambda i: (i, 0)),
            pl.BlockSpec((rb, de), lambda i: (i, 0)),
            pl.BlockSpec((rb, din), lambda i: (i, 0)),
            pl.BlockSpec((din, dout), lambda i: (0, 0)),
            pl.BlockSpec((de, dout), lambda i: (0, 0)),
            pl.BlockSpec((din, dout), lambda i: (0, 0)),
            pl.BlockSpec((dout, dout), lambda i: (0, 0)),
            pl.BlockSpec((1, dout), lambda i: (0, 0)),
            pl.BlockSpec((1, dout), lambda i: (0, 0)),
        ],
        out_specs=pl.BlockSpec((rb, dout), lambda i: (i, 0)),
        out_shape=jax.ShapeDtypeStruct((n_nodes, dout), jnp.float32),
    )(a0, a1, b_part, d_part, nf, wmh_t, wme_t, wa1_t, wa2_t, b_msg, b_apply)


def kernel(nfeats, efeats, edge_index, W_msg, b_msg, W_apply, b_apply):
    n_nodes = nfeats.shape[0]
    din = nfeats.shape[2]
    de = efeats.shape[2]
    dout = W_msg.shape[0]
    dh = din // 2

    nf = nfeats.reshape(n_nodes, din)
    ef = efeats.reshape(efeats.shape[0], de)
    src = edge_index[0]
    dst = edge_index[1]

    wmh_t = W_msg[:, :din].T
    wme_t = W_msg[:, din:].T
    wa1_t = W_apply[:, :din].T
    wa2_t = W_apply[:, din:].T

    a0, a1, b_part, d_part = _sc_accumulate(
        src, dst, nf[:, :dh], nf[:, dh:], ef, n_nodes)
    out = _tc_finish(a0, a1, b_part, d_part, nf, wmh_t, wme_t, wa1_t, wa2_t,
                     b_msg.reshape(1, dout), b_apply.reshape(1, dout), n_nodes)
    return out.reshape(n_nodes, 1, dout)


# speedup vs baseline: 6.9797x; 1.0019x over previous
import functools

import jax
import jax.numpy as jnp
from jax import lax
from jax.experimental import pallas as pl
from jax.experimental.pallas import tpu as pltpu
from jax.experimental.pallas import tpu_sc as plsc

_NC = 2
_NS = 16


def _sc_accumulate(src, dst, nf0, nf1, ef, n_nodes):
    e_total = src.shape[0]
    dh = nf0.shape[1]
    de = ef.shape[1]
    ept = e_total // _NS
    ch = 80
    sb = 5
    rows_pt = ept // ch
    nsc = rows_pt // sb
    rpt = (n_nodes // (8 * _NS)) * 8
    rem = n_nodes - _NS * rpt
    rem0 = _NS * rpt
    nst = 6
    spt = rpt // nst

    mesh = plsc.VectorSubcoreMesh(
        core_axis_name="c", subcore_axis_name="s",
        num_cores=_NC, num_subcores=_NS)

    @functools.partial(
        pl.kernel,
        out_type=[
            jax.ShapeDtypeStruct((n_nodes, dh), jnp.float32),
            jax.ShapeDtypeStruct((n_nodes, dh), jnp.float32),
            jax.ShapeDtypeStruct((n_nodes, de), jnp.float32),
            jax.ShapeDtypeStruct((n_nodes, de), jnp.float32),
        ],
        mesh=mesh,
        compiler_params=pltpu.CompilerParams(use_tc_tiling_on_sc=False),
        scratch_types=[
            pltpu.VMEM_SHARED((n_nodes, dh), jnp.float32),
            pltpu.VMEM_SHARED((n_nodes, de), jnp.float32),
            pltpu.VMEM((2, sb * ch), jnp.int32),
            pltpu.VMEM((2, sb * ch), jnp.int32),
            pltpu.VMEM((2, sb, ch), jnp.int32),
            pltpu.VMEM((2, sb, ch, dh), jnp.float32),
            pltpu.VMEM((2, sb * ch, de), jnp.float32),
            pltpu.VMEM((ch, de), jnp.float32),
            pltpu.VMEM((spt, dh), jnp.float32),
            pltpu.VMEM((spt, de), jnp.float32),
            pltpu.VMEM((rem, dh), jnp.float32),
            pltpu.VMEM((rem, de), jnp.float32),
            pltpu.SemaphoreType.DMA,
            pltpu.SemaphoreType.DMA,
            pltpu.SemaphoreType.DMA,
            pltpu.SemaphoreType.DMA,
            pltpu.SemaphoreType.DMA,
        ],
    )
    def sc_kernel(src_h, dst_h, nf0_h, nf1_h, ef_h, za_h, zb_h,
                  a0_out, a1_out, b_out, d_out,
                  a_sh, bd_sh,
                  src_i, dst_i, dst_s, rows_v, val_v, ones_v, sta, stb,
                  exa, exb,
                  sem_i0, sem_i1, sem_g, sem_s0, sem_s1):
        cid = lax.axis_index("c")
        sid = lax.axis_index("s")

        r0 = sid * rpt
        pltpu.sync_copy(za_h, sta)
        pltpu.sync_copy(zb_h, stb)
        for k in range(nst):
            pltpu.sync_copy(sta, a_sh.at[pl.ds(r0 + k * spt, spt)])
            pltpu.sync_copy(stb, bd_sh.at[pl.ds(r0 + k * spt, spt)])
        if rem:
            @pl.when(sid == 0)
            def _zero_rem():
                pltpu.sync_copy(za_h.at[pl.ds(0, rem)], exa)
                pltpu.sync_copy(zb_h.at[pl.ds(0, rem)], exb)
                pltpu.sync_copy(exa, a_sh.at[pl.ds(rem0, rem)])
                pltpu.sync_copy(exb, bd_sh.at[pl.ds(rem0, rem)])
        plsc.subcore_barrier()

        rowbase = sid * rows_pt

        def run_edges(nf_h, load_ef):
            if not load_ef:
                onehot = jnp.where(lax.iota(jnp.int32, de) == 0,
                                   jnp.float32(1.0), jnp.float32(0.0))

                def init_ones(i, carry):
                    ones_v[i, :] = onehot
                    return carry
                lax.fori_loop(0, ch, init_ones, 0)

            sem_i = (sem_i0, sem_i1)
            sem_s = (sem_s0, sem_s1)

            def idx_copies(row0, p, make_only):
                mk = pltpu.make_async_copy
                e0 = pl.multiple_of(row0 * ch, 8)
                ds = [mk(src_h.at[pl.ds(e0, sb * ch)], src_i.at[p],
                         sem_i[p]),
                      mk(dst_h.at[pl.ds(e0, sb * ch)], dst_i.at[p],
                         sem_i[p])]
                if not make_only:
                    for d in ds:
                        d.start()
                return ds

            def scatter_copies(p, make_only):
                mk = pltpu.make_async_copy
                ds = []
                for b in range(sb):
                    ds.append(mk(rows_v.at[p, b], a_sh.at[dst_s.at[p, b]],
                                 sem_s[p]))
                    vsrc = (val_v.at[p, pl.ds(b * ch, ch)] if load_ef
                            else ones_v)
                    ds.append(mk(vsrc, bd_sh.at[dst_s.at[p, b]], sem_s[p]))
                if not make_only:
                    for d in ds:
                        d.start(add=True)
                return ds

            idx_copies(rowbase, 0, False)
            idx_copies(rowbase + sb, 1, False)

            def section(s, p):
                row0 = rowbase + s * sb
                @pl.when(s >= 2)
                def _drain_scatters():
                    for d in scatter_copies(p, True):
                        d.wait()
                for d in idx_copies(row0, p, True):
                    d.wait()
                for b in range(sb):
                    for j in range(ch // 16):
                        dst_s[p, b, pl.ds(j * 16, 16)] = (
                            dst_i[p, pl.ds(b * ch + j * 16, 16)])
                gd = [pltpu.async_copy(
                          nf_h.at[src_i.at[p, pl.ds(b * ch, ch)]],
                          rows_v.at[p, b], sem_g)
                      for b in range(sb)]
                if load_ef:
                    gd.append(pltpu.async_copy(
                        ef_h.at[pl.ds(row0 * ch, sb * ch)], val_v.at[p],
                        sem_g))
                @pl.when(s < nsc - 1)
                def _prefetch():
                    idx_copies(row0 + sb, 1 - p, False)

                for d in gd:
                    d.wait()
                scatter_copies(p, False)

            def pairbody(k, carry):
                section(2 * k, 0)
                section(2 * k + 1, 1)
                return carry
            lax.fori_loop(0, nsc // 2, pairbody, 0)
            for p in range(2):
                for d in scatter_copies(p, True):
                    d.wait()

        @pl.when(cid == 0)
        def _core0():
            run_edges(nf0_h, True)

        @pl.when(cid == 1)
        def _core1():
            run_edges(nf1_h, False)

        plsc.subcore_barrier()

        for k in range(nst):
            rk = r0 + k * spt
            pltpu.sync_copy(a_sh.at[pl.ds(rk, spt)], sta)
            pltpu.sync_copy(bd_sh.at[pl.ds(rk, spt)], stb)

            @pl.when(cid == 0)
            def _w0():
                pltpu.sync_copy(sta, a0_out.at[pl.ds(rk, spt)])
                pltpu.sync_copy(stb, b_out.at[pl.ds(rk, spt)])

            @pl.when(cid == 1)
            def _w1():
                pltpu.sync_copy(sta, a1_out.at[pl.ds(rk, spt)])
                pltpu.sync_copy(stb, d_out.at[pl.ds(rk, spt)])

        if rem:
            @pl.when(sid == 0)
            def _write_rem():
                pltpu.sync_copy(a_sh.at[pl.ds(rem0, rem)], exa)
                pltpu.sync_copy(bd_sh.at[pl.ds(rem0, rem)], exb)

                @pl.when(cid == 0)
                def _wbr():
                    pltpu.sync_copy(exa, a0_out.at[pl.ds(rem0, rem)])
                    pltpu.sync_copy(exb, b_out.at[pl.ds(rem0, rem)])

                @pl.when(cid == 1)
                def _wdr():
                    pltpu.sync_copy(exa, a1_out.at[pl.ds(rem0, rem)])
                    pltpu.sync_copy(exb, d_out.at[pl.ds(rem0, rem)])

    za = jnp.zeros((spt, dh), jnp.float32)
    zb = jnp.zeros((spt, de), jnp.float32)
    return sc_kernel(src, dst, nf0, nf1, ef, za, zb)


def _tc_finish_body(a0_ref, a1_ref, b_ref, d_ref, nf_ref, wmh_ref, wme_ref,
                    wa1_ref, wa2_ref, bm_ref, ba_ref, o_ref):
    hi = jax.lax.Precision.HIGHEST
    dh = a0_ref.shape[1]
    deg = jnp.sum(d_ref[...], axis=1, keepdims=True)
    msum = (jnp.dot(a0_ref[...], wmh_ref[:dh], precision=hi,
                    preferred_element_type=jnp.float32)
            + jnp.dot(a1_ref[...], wmh_ref[dh:], precision=hi,
                      preferred_element_type=jnp.float32)
            + jnp.dot(b_ref[...], wme_ref[...], precision=hi,
                      preferred_element_type=jnp.float32)
            + deg * bm_ref[...])
    h_neigh = msum / jnp.maximum(deg, 1.0)
    h = (jnp.dot(nf_ref[...], wa1_ref[...], precision=hi,
                 preferred_element_type=jnp.float32)
         + jnp.dot(h_neigh, wa2_ref[...], precision=hi,
                   preferred_element_type=jnp.float32)
         + ba_ref[...])
    o_ref[...] = jnp.maximum(h, 0.0)


def _tc_finish(a0, a1, b_part, d_part, nf, wmh_t, wme_t, wa1_t, wa2_t,
               b_msg, b_apply, n_nodes):
    din = nf.shape[1]
    dh = a0.shape[1]
    de = b_part.shape[1]
    dout = wmh_t.shape[1]
    rb = 1000
    grid = (n_nodes // rb,)
    return pl.pallas_call(
        _tc_finish_body,
        grid=grid,
        in_specs=[
            pl.BlockSpec((rb, dh), lambda i: (i, 0)),
            pl.BlockSpec((rb, dh), lambda i: (i, 0)),
            pl.BlockSpec((rb, de), lambda i: (i, 0)),
            pl.BlockSpec((rb, de), lambda i: (i, 0)),
            pl.BlockSpec((rb, din), lambda i: (i, 0)),
            pl.BlockSpec((din, dout), lambda i: (0, 0)),
            pl.BlockSpec((de, dout), lambda i: (0, 0)),
            pl.BlockSpec((din, dout), lambda i: (0, 0)),
            pl.BlockSpec((dout, dout), lambda i: (0, 0)),
            pl.BlockSpec((1, dout), lambda i: (0, 0)),
            pl.BlockSpec((1, dout), lambda i: (0, 0)),
        ],
        out_specs=pl.BlockSpec((rb, dout), lambda i: (i, 0)),
        out_shape=jax.ShapeDtypeStruct((n_nodes, dout), jnp.float32),
    )(a0, a1, b_part, d_part, nf, wmh_t, wme_t, wa1_t, wa2_t, b_msg, b_apply)


def kernel(nfeats, efeats, edge_index, W_msg, b_msg, W_apply, b_apply):
    n_nodes = nfeats.shape[0]
    din = nfeats.shape[2]
    de = efeats.shape[2]
    dout = W_msg.shape[0]
    dh = din // 2

    nf = nfeats.reshape(n_nodes, din)
    ef = efeats.reshape(efeats.shape[0], de)
    src = edge_index[0]
    dst = edge_index[1]

    wmh_t = W_msg[:, :din].T
    wme_t = W_msg[:, din:].T
    wa1_t = W_apply[:, :din].T
    wa2_t = W_apply[:, din:].T

    a0, a1, b_part, d_part = _sc_accumulate(
        src, dst, nf[:, :dh], nf[:, dh:], ef, n_nodes)
    out = _tc_finish(a0, a1, b_part, d_part, nf, wmh_t, wme_t, wa1_t, wa2_t,
                     b_msg.reshape(1, dout), b_apply.reshape(1, dout), n_nodes)
    return out.reshape(n_nodes, 1, dout)
